# Initial kernel scaffold; baseline (speedup 1.0000x reference)
import jax
import jax.numpy as jnp
from jax.experimental import pallas as pl


def kernel(x_feature, y_feature, abs_v_x, rel_v_x, abs_v_y, rel_v_y, abs_t_x, rel_t_x, abs_t_y, rel_t_y):
    raise NotImplementedError("write your pallas kernel here")


# speedup vs baseline: 3.5987x; 3.5987x over previous
import functools
import math

import jax
import jax.numpy as jnp
import numpy as np
from jax import lax
from jax.experimental import pallas as pl
from jax.experimental.pallas import tpu as pltpu
from jax.experimental.pallas import tpu_sc as plsc

B = 1024
S = 200
MAX2D = 1024
CS = 96
D = 768
MAXLEN = 512
NCHUNK = 8
ROWW = 2 * CS
VROWS = 3 * MAX2D
TROWS = VROWS + 5 * 2 * MAX2D

NW = 32
BPW = B // NW
W = 8
NSB = S // W
NIDX = W * NCHUNK
L = 16


def _pe_table(seq_len, d_model):
    position = np.arange(MAXLEN)[:, None].astype(np.float32)
    div_term = np.exp(np.arange(0, d_model, 2).astype(np.float32)
                      * (-math.log(10000.0) / d_model))
    pe = np.zeros((MAXLEN, d_model), dtype=np.float32)
    pe[:, 0::2] = np.sin(position * div_term)
    pe[:, 1::2] = np.cos(position * div_term)
    return pe[:seq_len]


def _sc_body(tx, ty, xf, yf, pe, ov, ot,
             raw_x, raw_y, idx_x, idx_y, rows_x, rows_y,
             pe_v, ov_s, ot_s, sem_x, sem_y):
    wid = lax.axis_index("s") * 2 + lax.axis_index("c")
    b0 = wid * BPW

    ch = lax.iota(jnp.int32, L) & 7
    is_abs = ch < 3
    lo = jnp.where(is_abs, 0, -MAX2D)
    off = jnp.where(is_abs, ch * MAX2D, ch * (2 * MAX2D) - 2 * MAX2D)
    hi = MAX2D - 1

    def do_block(b, s0):
        fbase = b * (S * NCHUNK) + s0 * NCHUNK
        pltpu.sync_copy(xf.at[pl.ds(fbase, NIDX)], raw_x)
        pltpu.sync_copy(yf.at[pl.ds(fbase, NIDX)], raw_y)
        for k in range(NIDX // L):
            sl = pl.ds(k * L, L)
            idx_x[sl] = jnp.minimum(jnp.maximum(raw_x[sl], lo), hi) + off
            idx_y[sl] = jnp.minimum(jnp.maximum(raw_y[sl], lo), hi) + off
        gx = pltpu.async_copy(tx.at[idx_x], rows_x, sem_x)
        gy = pltpu.async_copy(ty.at[idx_y], rows_y, sem_y)
        gx.wait()
        gy.wait()

        def pos(p, c):
            for i in range(NCHUNK):
                r = p * NCHUNK + i
                for jj in range(CS // L):
                    cv = pl.ds(jj * L, L)
                    ct = pl.ds(CS + jj * L, L)
                    co = pl.ds(i * CS + jj * L, L)
                    pv = pe_v[p, co]
                    ov_s[p, co] = rows_x[r, cv] + rows_y[r, cv] + pv
                    ot_s[p, co] = rows_x[r, ct] + rows_y[r, ct] + pv
            return c

        lax.fori_loop(0, W, pos, 0)
        pltpu.sync_copy(ov_s, ov.at[b, pl.ds(s0, W)])
        pltpu.sync_copy(ot_s, ot.at[b, pl.ds(s0, W)])

    def sblk_iter(sb, c):
        s0 = sb * W
        pltpu.sync_copy(pe.at[pl.ds(s0, W)], pe_v)

        def b_iter(bi, c2):
            do_block(b0 + bi, s0)
            return c2

        lax.fori_loop(0, BPW, b_iter, 0)
        return c

    lax.fori_loop(0, NSB, sblk_iter, 0)


def kernel(x_feature, y_feature, abs_v_x, rel_v_x, abs_v_y, rel_v_y,
           abs_t_x, rel_t_x, abs_t_y, rel_t_y):
    tx = jnp.concatenate(
        [jnp.concatenate([abs_v_x.reshape(VROWS, CS),
                          rel_v_x.reshape(TROWS - VROWS, CS)], axis=0),
         jnp.concatenate([abs_t_x.reshape(VROWS, CS),
                          rel_t_x.reshape(TROWS - VROWS, CS)], axis=0)],
        axis=1)
    ty = jnp.concatenate(
        [jnp.concatenate([abs_v_y.reshape(VROWS, CS),
                          rel_v_y.reshape(TROWS - VROWS, CS)], axis=0),
         jnp.concatenate([abs_t_y.reshape(VROWS, CS),
                          rel_t_y.reshape(TROWS - VROWS, CS)], axis=0)],
        axis=1)
    xf = x_feature.reshape(B * S * NCHUNK)
    yf = y_feature.reshape(B * S * NCHUNK)
    pe = jnp.asarray(_pe_table(S, D))

    mesh = plsc.VectorSubcoreMesh(core_axis_name="c", subcore_axis_name="s")
    f = functools.partial(
        pl.kernel,
        out_type=(jax.ShapeDtypeStruct((B, S, D), jnp.float32),
                  jax.ShapeDtypeStruct((B, S, D), jnp.float32)),
        mesh=mesh,
        compiler_params=pltpu.CompilerParams(use_tc_tiling_on_sc=False),
        scratch_types=[
            pltpu.VMEM((NIDX,), jnp.int32),
            pltpu.VMEM((NIDX,), jnp.int32),
            pltpu.VMEM((NIDX,), jnp.int32),
            pltpu.VMEM((NIDX,), jnp.int32),
            pltpu.VMEM((NIDX, ROWW), jnp.float32),
            pltpu.VMEM((NIDX, ROWW), jnp.float32),
            pltpu.VMEM((W, D), jnp.float32),
            pltpu.VMEM((W, D), jnp.float32),
            pltpu.VMEM((W, D), jnp.float32),
            pltpu.SemaphoreType.DMA,
            pltpu.SemaphoreType.DMA,
        ],
    )(_sc_body)
    v_bar, t_bar = f(tx, ty, xf, yf, pe)
    return (v_bar, t_bar)


# speedup vs baseline: 9.3257x; 2.5914x over previous
import functools
import math

import jax
import jax.numpy as jnp
import numpy as np
from jax import lax
from jax.experimental import pallas as pl
from jax.experimental.pallas import tpu as pltpu
from jax.experimental.pallas import tpu_sc as plsc

B = 1024
S = 200
MAX2D = 1024
CS = 96
D = 768
NCHUNK = 8
TROWS = 3 * MAX2D + 5 * 2 * MAX2D

NW = 32
BPW = B // NW
W = 8
NSB = S // W
NIDX = W * NCHUNK
L = 16
FROW = S * NCHUNK


def _pe_table(seq_len, d_model):
    position = np.arange(seq_len)[:, None].astype(np.float32)
    div_term = np.exp(np.arange(0, d_model, 2).astype(np.float32)
                      * (-math.log(10000.0) / d_model))
    pe = np.zeros((seq_len, d_model), dtype=np.float32)
    pe[:, 0::2] = np.sin(position * div_term)
    pe[:, 1::2] = np.cos(position * div_term)
    return pe


def _sc_body(tvx, ttx, tvy, tty, xf, yf, pe, ov, ot,
             raw_x, raw_y, idx_x, idx_y, stv, stt, bvy, bty, pe_v,
             s_rx, s_ry, s_gvx, s_gtx, s_gvy, s_gty):
    wid = lax.axis_index("s") * 2 + lax.axis_index("c")
    b0 = wid * BPW

    ch = lax.iota(jnp.int32, L) & 7
    is_abs = ch < 3
    lo = jnp.where(is_abs, 0, -MAX2D)
    off = jnp.where(is_abs, ch * MAX2D, ch * (2 * MAX2D) - 2 * MAX2D)
    hi = MAX2D - 1

    def fbase(n, s0):
        return (b0 + n) * FROW + s0 * NCHUNK

    def raw_copy(n, s0, k):
        fb = fbase(n, s0)
        cx = pltpu.make_async_copy(xf.at[pl.ds(fb, NIDX)], raw_x.at[k], s_rx[k])
        cy = pltpu.make_async_copy(yf.at[pl.ds(fb, NIDX)], raw_y.at[k], s_ry[k])
        return cx, cy

    def fire_raw(n, s0, k):
        cx, cy = raw_copy(n, s0, k)
        cx.start()
        cy.start()

    def wait_raw(n, s0, k):
        cx, cy = raw_copy(n, s0, k)
        cx.wait()
        cy.wait()

    def gathers(k):
        return (
            pltpu.make_async_copy(tvx.at[idx_x.at[k]], stv.at[k], s_gvx[k]),
            pltpu.make_async_copy(ttx.at[idx_x.at[k]], stt.at[k], s_gtx[k]),
            pltpu.make_async_copy(tvy.at[idx_y.at[k]], bvy.at[k], s_gvy[k]),
            pltpu.make_async_copy(tty.at[idx_y.at[k]], bty.at[k], s_gty[k]),
        )

    def fire_gathers(k):
        for c in gathers(k):
            c.start()

    def wait_gathers(k):
        for c in gathers(k):
            c.wait()

    def idx_compute(k):
        for g in range(NIDX // L):
            sl = pl.ds(g * L, L)
            idx_x[k, sl] = jnp.minimum(jnp.maximum(raw_x[k, sl], lo), hi) + off
            idx_y[k, sl] = jnp.minimum(jnp.maximum(raw_y[k, sl], lo), hi) + off

    def compute_out(n, s0, k):
        def row(r, c):
            for jj in range(CS // L):
                sl = pl.ds(jj * L, L)
                pv = pe_v[r, sl]
                plsc.addupdate(stv.at[k, r, sl], bvy[k, r, sl] + pv)
                plsc.addupdate(stt.at[k, r, sl], bty[k, r, sl] + pv)
            return c

        lax.fori_loop(0, NIDX, row, 0)
        obase = ((b0 + n) * S + s0) * NCHUNK
        pltpu.sync_copy(stv.at[k], ov.at[pl.ds(obase, NIDX)])
        pltpu.sync_copy(stt.at[k], ot.at[pl.ds(obase, NIDX)])

    def prep_and_fire(n, s0, k):
        wait_raw(n, s0, k)
        idx_compute(k)
        fire_gathers(k)

    def stage(n, s0, cur, nxt):
        prep_and_fire(n + 1, s0, nxt)
        fire_raw(jnp.minimum(n + 2, BPW - 1), s0, cur)
        wait_gathers(cur)
        compute_out(n, s0, cur)

    def sblk_iter(sb, c):
        s0 = sb * W
        pltpu.sync_copy(pe.at[pl.ds(s0 * NCHUNK, NIDX)], pe_v)
        fire_raw(0, s0, 0)
        wait_raw(0, s0, 0)
        idx_compute(0)
        fire_gathers(0)
        fire_raw(1, s0, 1)

        def pair(i, c2):
            n = i * 2
            stage(n, s0, 0, 1)
            stage(n + 1, s0, 1, 0)
            return c2

        lax.fori_loop(0, BPW // 2 - 1, pair, 0)
        stage(BPW - 2, s0, 0, 1)
        wait_raw(BPW - 1, s0, 0)
        wait_gathers(1)
        compute_out(BPW - 1, s0, 1)
        return c

    lax.fori_loop(0, NSB, sblk_iter, 0)


def kernel(x_feature, y_feature, abs_v_x, rel_v_x, abs_v_y, rel_v_y,
           abs_t_x, rel_t_x, abs_t_y, rel_t_y):
    def table(a, r):
        return jnp.concatenate([a.reshape(3 * MAX2D, CS),
                                r.reshape(10 * MAX2D, CS)], axis=0)

    tvx = table(abs_v_x, rel_v_x)
    ttx = table(abs_t_x, rel_t_x)
    tvy = table(abs_v_y, rel_v_y)
    tty = table(abs_t_y, rel_t_y)
    xf = x_feature.reshape(B * FROW)
    yf = y_feature.reshape(B * FROW)
    pe = jnp.asarray(_pe_table(S, D)).reshape(S * NCHUNK, CS)

    mesh = plsc.VectorSubcoreMesh(core_axis_name="c", subcore_axis_name="s")
    f = functools.partial(
        pl.kernel,
        out_type=(jax.ShapeDtypeStruct((B * S * NCHUNK, CS), jnp.float32),
                  jax.ShapeDtypeStruct((B * S * NCHUNK, CS), jnp.float32)),
        mesh=mesh,
        compiler_params=pltpu.CompilerParams(use_tc_tiling_on_sc=False),
        scratch_types=[
            pltpu.VMEM((2, NIDX), jnp.int32),
            pltpu.VMEM((2, NIDX), jnp.int32),
            pltpu.VMEM((2, NIDX), jnp.int32),
            pltpu.VMEM((2, NIDX), jnp.int32),
            pltpu.VMEM((2, NIDX, CS), jnp.float32),
            pltpu.VMEM((2, NIDX, CS), jnp.float32),
            pltpu.VMEM((2, NIDX, CS), jnp.float32),
            pltpu.VMEM((2, NIDX, CS), jnp.float32),
            pltpu.VMEM((NIDX, CS), jnp.float32),
            [pltpu.SemaphoreType.DMA] * 2,
            [pltpu.SemaphoreType.DMA] * 2,
            [pltpu.SemaphoreType.DMA] * 2,
            [pltpu.SemaphoreType.DMA] * 2,
            [pltpu.SemaphoreType.DMA] * 2,
            [pltpu.SemaphoreType.DMA] * 2,
        ],
    )(_sc_body)
    v_bar, t_bar = f(tvx, ttx, tvy, tty, xf, yf, pe)
    return (v_bar.reshape(B, S, D), t_bar.reshape(B, S, D))


# speedup vs baseline: 10.3004x; 1.1045x over previous
import functools
import math

import jax
import jax.numpy as jnp
import numpy as np
from jax import lax
from jax.experimental import pallas as pl
from jax.experimental.pallas import tpu as pltpu
from jax.experimental.pallas import tpu_sc as plsc

B = 1024
S = 200
MAX2D = 1024
CS = 96
D = 768
NCHUNK = 8
TROWS = 3 * MAX2D + 5 * 2 * MAX2D
ROWW = 2 * CS

NW = 32
BPW = B // NW
W = 8
NSB = S // W
NIDX = W * NCHUNK
L = 16


def _pe_table(seq_len, d_model):
    position = np.arange(seq_len)[:, None].astype(np.float32)
    div_term = np.exp(np.arange(0, d_model, 2).astype(np.float32)
                      * (-math.log(10000.0) / d_model))
    pe = np.zeros((seq_len, d_model), dtype=np.float32)
    pe[:, 0::2] = np.sin(position * div_term)
    pe[:, 1::2] = np.cos(position * div_term)
    return pe


def _ileave(t):
    r, c = t.shape
    return t.reshape(r, c // 32, 2, 16).transpose(0, 1, 3, 2).reshape(r, c)


def _sc_body(tx, ty, xy, pe, ov, ot,
             raw, idx_x, idx_y, bx, by, stv, stt, pe_v,
             s_r, s_gx, s_gy, s_ov, s_ot):
    wid = lax.axis_index("s") * 2 + lax.axis_index("c")
    b0 = wid * BPW

    ch = lax.iota(jnp.int32, L) & 7
    is_abs = ch < 3
    lo = jnp.where(is_abs, 0, -MAX2D)
    off = jnp.where(is_abs, ch * MAX2D, ch * (2 * MAX2D) - 2 * MAX2D)
    hi = MAX2D - 1

    def raw_copy(n, sb, k):
        fb = ((b0 + n) * NSB + sb) * (2 * NIDX)
        return pltpu.make_async_copy(xy.at[pl.ds(fb, 2 * NIDX)], raw.at[k],
                                     s_r[k])

    def gathers(k):
        return (
            pltpu.make_async_copy(tx.at[idx_x.at[k]], bx.at[k], s_gx[k]),
            pltpu.make_async_copy(ty.at[idx_y.at[k]], by.at[k], s_gy[k]),
        )

    def fire_gathers(k):
        for c in gathers(k):
            c.start()

    def wait_gathers(k):
        for c in gathers(k):
            c.wait()

    def out_copies(n, sb, k):
        obase = ((b0 + n) * S + sb * W) * NCHUNK
        return (
            pltpu.make_async_copy(stv.at[k], ov.at[pl.ds(obase, NIDX)],
                                  s_ov[k]),
            pltpu.make_async_copy(stt.at[k], ot.at[pl.ds(obase, NIDX)],
                                  s_ot[k]),
        )

    def fire_outs(n, sb, k):
        for c in out_copies(n, sb, k):
            c.start()

    def wait_outs(k):
        for c in out_copies(0, 0, k):
            c.wait()

    def idx_compute(k):
        for g in range(NIDX // L):
            sl = pl.ds(g * L, L)
            sly = pl.ds(NIDX + g * L, L)
            idx_x[k, sl] = jnp.minimum(jnp.maximum(raw[k, sl], lo), hi) + off
            idx_y[k, sl] = jnp.minimum(jnp.maximum(raw[k, sly], lo), hi) + off

    def compute(n, sb, k):
        def row(r, c):
            for gp in range(3):
                slp = pl.ds(gp * 32, 32)
                pe1, pe2 = plsc.unpack(pe_v[r, slp],
                                       format=plsc.PackFormat.INTERLEAVED,
                                       preferred_element_type=jnp.float32)
                for half, st in ((0, stv), (1, stt)):
                    slg = pl.ds(half * CS + gp * 32, 32)
                    xa, xb = plsc.unpack(bx[k, r, slg],
                                         format=plsc.PackFormat.INTERLEAVED,
                                         preferred_element_type=jnp.float32)
                    ya, yb = plsc.unpack(by[k, r, slg],
                                         format=plsc.PackFormat.INTERLEAVED,
                                         preferred_element_type=jnp.float32)
                    st[k, r, pl.ds(gp * 32, L)] = xa + ya + pe1
                    st[k, r, pl.ds(gp * 32 + L, L)] = xb + yb + pe2
            return c

        lax.fori_loop(0, NIDX, row, 0)
        fire_outs(n, sb, k)

    def stage(n, sb, cur, nxt, wait_out_cur):
        raw_copy(n + 1, sb, nxt).wait()
        idx_compute(nxt)
        fire_gathers(nxt)
        raw_copy(jnp.minimum(n + 2, BPW - 1), sb, cur).start()
        wait_gathers(cur)
        if wait_out_cur:
            wait_outs(cur)
        compute(n, sb, cur)

    def sblk_iter(sb, c):
        pltpu.sync_copy(pe.at[pl.ds(sb * NIDX, NIDX)], pe_v)
        raw_copy(0, sb, 0).start()
        raw_copy(0, sb, 0).wait()
        idx_compute(0)
        fire_gathers(0)
        raw_copy(1, sb, 1).start()
        stage(0, sb, 0, 1, False)
        stage(1, sb, 1, 0, False)

        def pair(i, c2):
            n = i * 2
            stage(n, sb, 0, 1, True)
            stage(n + 1, sb, 1, 0, True)
            return c2

        lax.fori_loop(1, BPW // 2 - 1, pair, 0)
        stage(BPW - 2, sb, 0, 1, True)
        raw_copy(BPW - 1, sb, 0).wait()
        wait_gathers(1)
        wait_outs(1)
        compute(BPW - 1, sb, 1)
        wait_outs(0)
        wait_outs(1)
        return c

    lax.fori_loop(0, NSB, sblk_iter, 0)


def kernel(x_feature, y_feature, abs_v_x, rel_v_x, abs_v_y, rel_v_y,
           abs_t_x, rel_t_x, abs_t_y, rel_t_y):
    def table(av, rv, at, rt):
        v = jnp.concatenate([av.reshape(3 * MAX2D, CS),
                             rv.reshape(10 * MAX2D, CS)], axis=0)
        t = jnp.concatenate([at.reshape(3 * MAX2D, CS),
                             rt.reshape(10 * MAX2D, CS)], axis=0)
        return _ileave(jnp.concatenate([v, t], axis=1)).astype(jnp.bfloat16)

    tx = table(abs_v_x, rel_v_x, abs_t_x, rel_t_x)
    ty = table(abs_v_y, rel_v_y, abs_t_y, rel_t_y)
    xy = jnp.stack([x_feature.reshape(B, NSB, NIDX),
                    y_feature.reshape(B, NSB, NIDX)], axis=2)
    xy = xy.reshape(B * NSB * 2 * NIDX)
    pe = _ileave(jnp.asarray(_pe_table(S, D)).reshape(S * NCHUNK, CS))
    pe = pe.astype(jnp.bfloat16)

    mesh = plsc.VectorSubcoreMesh(core_axis_name="c", subcore_axis_name="s")
    f = functools.partial(
        pl.kernel,
        out_type=(jax.ShapeDtypeStruct((B * S * NCHUNK, CS), jnp.float32),
                  jax.ShapeDtypeStruct((B * S * NCHUNK, CS), jnp.float32)),
        mesh=mesh,
        compiler_params=pltpu.CompilerParams(use_tc_tiling_on_sc=False,
                                             needs_layout_passes=False),
        scratch_types=[
            pltpu.VMEM((2, 2 * NIDX), jnp.int32),
            pltpu.VMEM((2, NIDX), jnp.int32),
            pltpu.VMEM((2, NIDX), jnp.int32),
            pltpu.VMEM((2, NIDX, ROWW), jnp.bfloat16),
            pltpu.VMEM((2, NIDX, ROWW), jnp.bfloat16),
            pltpu.VMEM((2, NIDX, CS), jnp.float32),
            pltpu.VMEM((2, NIDX, CS), jnp.float32),
            pltpu.VMEM((NIDX, CS), jnp.bfloat16),
            [pltpu.SemaphoreType.DMA] * 2,
            [pltpu.SemaphoreType.DMA] * 2,
            [pltpu.SemaphoreType.DMA] * 2,
            [pltpu.SemaphoreType.DMA] * 2,
            [pltpu.SemaphoreType.DMA] * 2,
        ],
    )(_sc_body)
    v_bar, t_bar = f(tx, ty, xy, pe)
    return (v_bar.reshape(B, S, D), t_bar.reshape(B, S, D))


# speedup vs baseline: 18.3988x; 1.7862x over previous
import functools
import math

import jax
import jax.numpy as jnp
import numpy as np
from jax import lax
from jax.experimental import pallas as pl
from jax.experimental.pallas import tpu as pltpu
from jax.experimental.pallas import tpu_sc as plsc

B = 1024
S = 200
MAX2D = 1024
CS = 96
D = 768
NCHUNK = 8
TROWS = 3 * MAX2D + 5 * 2 * MAX2D
TW = 128
PW = CS // 2

NW = 32
BPW = B // NW
W = 8
NSB = S // W
NIDX = W * NCHUNK
L = 16


def _pe_table(seq_len, d_model):
    position = np.arange(seq_len)[:, None].astype(np.float32)
    div_term = np.exp(np.arange(0, d_model, 2).astype(np.float32)
                      * (-math.log(10000.0) / d_model))
    pe = np.zeros((seq_len, d_model), dtype=np.float32)
    pe[:, 0::2] = np.sin(position * div_term)
    pe[:, 1::2] = np.cos(position * div_term)
    return pe


def _ileave(t):
    r, c = t.shape
    return t.reshape(r, c // 32, 2, 16).transpose(0, 1, 3, 2).reshape(r, c)


def _pack_words(t):
    r, c = t.shape
    bf = _ileave(t).astype(jnp.bfloat16).reshape(r, c // 2, 2)
    return jax.lax.bitcast_convert_type(bf, jnp.int32)


def _sc_body(tx, ty, xy, pe, ov, ot,
             raw, idx_x, idx_y, bx, by, stv, stt, pe_v,
             s_r, s_gx, s_gy, s_ov, s_ot):
    wid = lax.axis_index("s") * 2 + lax.axis_index("c")
    b0 = wid * BPW

    ch = lax.iota(jnp.int32, L) & 7
    is_abs = ch < 3
    lo = jnp.where(is_abs, 0, -MAX2D)
    off = jnp.where(is_abs, ch * MAX2D, ch * (2 * MAX2D) - 2 * MAX2D)
    hi = MAX2D - 1

    def raw_copy(n, sb, k):
        fb = ((b0 + n) * NSB + sb) * (2 * NIDX)
        return pltpu.make_async_copy(xy.at[pl.ds(fb, 2 * NIDX)], raw.at[k],
                                     s_r[k])

    def gathers(k):
        return (
            pltpu.make_async_copy(tx.at[idx_x.at[k]], bx.at[k], s_gx[k]),
            pltpu.make_async_copy(ty.at[idx_y.at[k]], by.at[k], s_gy[k]),
        )

    def fire_gathers(k):
        for c in gathers(k):
            c.start()

    def wait_gathers(k):
        for c in gathers(k):
            c.wait()

    def out_copies(n, sb, k):
        b = b0 + n
        s0 = sb * W
        return (
            pltpu.make_async_copy(stv.at[k], ov.at[b, pl.ds(s0, W)], s_ov[k]),
            pltpu.make_async_copy(stt.at[k], ot.at[b, pl.ds(s0, W)], s_ot[k]),
        )

    def fire_outs(n, sb, k):
        for c in out_copies(n, sb, k):
            c.start()

    def wait_outs(k):
        for c in out_copies(0, 0, k):
            c.wait()

    def idx_compute(k):
        for g in range(NIDX // L):
            sl = pl.ds(g * L, L)
            sly = pl.ds(NIDX + g * L, L)
            idx_x[k, sl] = jnp.minimum(jnp.maximum(raw[k, sl], lo), hi) + off
            idx_y[k, sl] = jnp.minimum(jnp.maximum(raw[k, sly], lo), hi) + off

    def compute(n, sb, k):
        @plsc.parallel_loop(0, W, 1, unroll=2)
        def pos(p):
            for i in range(NCHUNK):
                r = p * NCHUNK + i
                for gp in range(3):
                    wpe = plsc.bitcast(pe_v[r, pl.ds(gp * L, L)],
                                       jnp.bfloat16)
                    for half, st in ((0, stv), (1, stt)):
                        slg = pl.ds(half * PW + gp * L, L)
                        wx = plsc.bitcast(bx[k, r, slg], jnp.bfloat16)
                        wy = plsc.bitcast(by[k, r, slg], jnp.bfloat16)
                        sm = wx + wy + wpe
                        a, b2 = plsc.unpack(
                            sm, format=plsc.PackFormat.INTERLEAVED,
                            preferred_element_type=jnp.float32)
                        col = i * CS + gp * 32
                        st[k, p, pl.ds(col, L)] = a
                        st[k, p, pl.ds(col + L, L)] = b2

        del pos
        fire_outs(n, sb, k)

    def stage(n, sb, cur, nxt, wait_out_cur):
        raw_copy(n + 1, sb, nxt).wait()
        idx_compute(nxt)
        fire_gathers(nxt)
        raw_copy(jnp.minimum(n + 2, BPW - 1), sb, cur).start()
        wait_gathers(cur)
        if wait_out_cur:
            wait_outs(cur)
        compute(n, sb, cur)

    def sblk_iter(sb, c):
        pltpu.sync_copy(pe.at[pl.ds(sb * NIDX, NIDX)], pe_v)
        raw_copy(0, sb, 0).start()
        raw_copy(0, sb, 0).wait()
        idx_compute(0)
        fire_gathers(0)
        raw_copy(1, sb, 1).start()
        stage(0, sb, 0, 1, False)
        stage(1, sb, 1, 0, False)

        def pair(i, c2):
            n = i * 2
            stage(n, sb, 0, 1, True)
            stage(n + 1, sb, 1, 0, True)
            return c2

        lax.fori_loop(1, BPW // 2 - 1, pair, 0)
        stage(BPW - 2, sb, 0, 1, True)
        raw_copy(BPW - 1, sb, 0).wait()
        wait_gathers(1)
        wait_outs(1)
        compute(BPW - 1, sb, 1)
        wait_outs(0)
        wait_outs(1)
        return c

    lax.fori_loop(0, NSB, sblk_iter, 0)


def kernel(x_feature, y_feature, abs_v_x, rel_v_x, abs_v_y, rel_v_y,
           abs_t_x, rel_t_x, abs_t_y, rel_t_y):
    def table(av, rv, at, rt):
        v = jnp.concatenate([av.reshape(3 * MAX2D, CS),
                             rv.reshape(10 * MAX2D, CS)], axis=0)
        t = jnp.concatenate([at.reshape(3 * MAX2D, CS),
                             rt.reshape(10 * MAX2D, CS)], axis=0)
        w = _pack_words(jnp.concatenate([v, t], axis=1))
        return jnp.pad(w, ((0, 0), (0, TW - w.shape[1])))

    tx = table(abs_v_x, rel_v_x, abs_t_x, rel_t_x)
    ty = table(abs_v_y, rel_v_y, abs_t_y, rel_t_y)
    xy = jnp.stack([x_feature.reshape(B, NSB, NIDX),
                    y_feature.reshape(B, NSB, NIDX)], axis=2)
    xy = xy.reshape(B * NSB * 2 * NIDX)
    pe = _pack_words(jnp.asarray(_pe_table(S, D)).reshape(S * NCHUNK, CS))

    mesh = plsc.VectorSubcoreMesh(core_axis_name="c", subcore_axis_name="s")
    f = functools.partial(
        pl.kernel,
        out_type=(jax.ShapeDtypeStruct((B, S, D), jnp.float32),
                  jax.ShapeDtypeStruct((B, S, D), jnp.float32)),
        mesh=mesh,
        compiler_params=pltpu.CompilerParams(use_tc_tiling_on_sc=True,
                                             needs_layout_passes=False),
        scratch_types=[
            pltpu.VMEM((2, 2 * NIDX), jnp.int32),
            pltpu.VMEM((2, NIDX), jnp.int32),
            pltpu.VMEM((2, NIDX), jnp.int32),
            pltpu.VMEM((2, NIDX, TW), jnp.int32),
            pltpu.VMEM((2, NIDX, TW), jnp.int32),
            pltpu.VMEM((2, W, D), jnp.float32),
            pltpu.VMEM((2, W, D), jnp.float32),
            pltpu.VMEM((NIDX, PW), jnp.int32),
            [pltpu.SemaphoreType.DMA] * 2,
            [pltpu.SemaphoreType.DMA] * 2,
            [pltpu.SemaphoreType.DMA] * 2,
            [pltpu.SemaphoreType.DMA] * 2,
            [pltpu.SemaphoreType.DMA] * 2,
        ],
    )(_sc_body)
    v_bar, t_bar = f(tx, ty, xy, pe)
    return (v_bar, t_bar)


# speedup vs baseline: 20.3159x; 1.1042x over previous
import functools
import math

import jax
import jax.numpy as jnp
import numpy as np
from jax import lax
from jax.experimental import pallas as pl
from jax.experimental.pallas import tpu as pltpu
from jax.experimental.pallas import tpu_sc as plsc

B = 1024
S = 200
MAX2D = 1024
CS = 96
D = 768
NCHUNK = 8
TROWS = 3 * MAX2D + 5 * 2 * MAX2D
TW = 128
PW = CS // 2

NW = 32
BPW = B // NW
W = 8
NSB = S // W
NIDX = W * NCHUNK
L = 16


def _pe_table(seq_len, d_model):
    position = np.arange(seq_len)[:, None].astype(np.float32)
    div_term = np.exp(np.arange(0, d_model, 2).astype(np.float32)
                      * (-math.log(10000.0) / d_model))
    pe = np.zeros((seq_len, d_model), dtype=np.float32)
    pe[:, 0::2] = np.sin(position * div_term)
    pe[:, 1::2] = np.cos(position * div_term)
    return pe


def _ileave(t):
    r, c = t.shape
    return t.reshape(r, c // 32, 2, 16).transpose(0, 1, 3, 2).reshape(r, c)


def _pack_words(t):
    r, c = t.shape
    bf = _ileave(t).astype(jnp.bfloat16).reshape(r, c // 2, 2)
    return jax.lax.bitcast_convert_type(bf, jnp.int32)


def _sc_body(tx, ty, xy, pe, ov, ot,
             raw, idx_x, idx_y, bx, by, stv, stt, pe_v,
             s_r, s_gx, s_gy, s_ov, s_ot):
    wid = lax.axis_index("s") * 2 + lax.axis_index("c")
    b0 = wid * BPW

    ch = lax.iota(jnp.int32, L) & 7
    is_abs = ch < 3
    lo = jnp.where(is_abs, 0, -MAX2D)
    off = jnp.where(is_abs, ch * MAX2D, ch * (2 * MAX2D) - 2 * MAX2D)
    hi = MAX2D - 1

    def raw_copy(n, sb, k):
        fb = ((b0 + n) * NSB + sb) * (2 * NIDX)
        return pltpu.make_async_copy(xy.at[pl.ds(fb, 2 * NIDX)], raw.at[k],
                                     s_r[k])

    def gathers(k):
        return (
            pltpu.make_async_copy(tx.at[idx_x.at[k]], bx.at[k], s_gx[k]),
            pltpu.make_async_copy(ty.at[idx_y.at[k]], by.at[k], s_gy[k]),
        )

    def fire_gathers(k):
        for c in gathers(k):
            c.start()

    def wait_gathers(k):
        for c in gathers(k):
            c.wait()

    def out_copies(n, sb, k):
        b = b0 + n
        s0 = sb * W
        return (
            pltpu.make_async_copy(stv.at[k], ov.at[b, pl.ds(s0, W)], s_ov[k]),
            pltpu.make_async_copy(stt.at[k], ot.at[b, pl.ds(s0, W)], s_ot[k]),
        )

    def fire_outs(n, sb, k):
        for c in out_copies(n, sb, k):
            c.start()

    def wait_outs(k):
        for c in out_copies(0, 0, k):
            c.wait()

    def idx_compute(k):
        for g in range(NIDX // L):
            sl = pl.ds(g * L, L)
            sly = pl.ds(NIDX + g * L, L)
            idx_x[k, sl] = jnp.minimum(jnp.maximum(raw[k, sl], lo), hi) + off
            idx_y[k, sl] = jnp.minimum(jnp.maximum(raw[k, sly], lo), hi) + off

    def compute(n, sb, k):
        def pos(p, c):
            for i0 in (0, NCHUNK // 2):
                sums = []
                for i in range(i0, i0 + NCHUNK // 2):
                    r = p * NCHUNK + i
                    for gp in range(3):
                        wpe = plsc.bitcast(pe_v[r, pl.ds(gp * L, L)],
                                           jnp.bfloat16)
                        for half, st in ((0, stv), (1, stt)):
                            slg = pl.ds(half * PW + gp * L, L)
                            wx = plsc.bitcast(bx[k, r, slg], jnp.bfloat16)
                            wy = plsc.bitcast(by[k, r, slg], jnp.bfloat16)
                            sums.append((st, i * CS + gp * 32,
                                         wx + wy + wpe))
                for st, col, sm in sums:
                    a, b2 = plsc.unpack(
                        sm, format=plsc.PackFormat.INTERLEAVED,
                        preferred_element_type=jnp.float32)
                    st[k, p, pl.ds(col, L)] = a
                    st[k, p, pl.ds(col + L, L)] = b2
            return c

        lax.fori_loop(0, W, pos, 0)
        fire_outs(n, sb, k)

    def stage(n, sb, cur, nxt, wait_out_cur):
        raw_copy(n + 1, sb, nxt).wait()
        idx_compute(nxt)
        fire_gathers(nxt)
        raw_copy(jnp.minimum(n + 2, BPW - 1), sb, cur).start()
        wait_gathers(cur)
        if wait_out_cur:
            wait_outs(cur)
        compute(n, sb, cur)

    def sblk_iter(sb, c):
        pltpu.sync_copy(pe.at[pl.ds(sb * NIDX, NIDX)], pe_v)
        raw_copy(0, sb, 0).start()
        raw_copy(0, sb, 0).wait()
        idx_compute(0)
        fire_gathers(0)
        raw_copy(1, sb, 1).start()
        stage(0, sb, 0, 1, False)
        stage(1, sb, 1, 0, False)

        def pair(i, c2):
            n = i * 2
            stage(n, sb, 0, 1, True)
            stage(n + 1, sb, 1, 0, True)
            return c2

        lax.fori_loop(1, BPW // 2 - 1, pair, 0)
        stage(BPW - 2, sb, 0, 1, True)
        raw_copy(BPW - 1, sb, 0).wait()
        wait_gathers(1)
        wait_outs(1)
        compute(BPW - 1, sb, 1)
        wait_outs(0)
        wait_outs(1)
        return c

    lax.fori_loop(0, NSB, sblk_iter, 0)


def kernel(x_feature, y_feature, abs_v_x, rel_v_x, abs_v_y, rel_v_y,
           abs_t_x, rel_t_x, abs_t_y, rel_t_y):
    def table(av, rv, at, rt):
        v = jnp.concatenate([av.reshape(3 * MAX2D, CS),
                             rv.reshape(10 * MAX2D, CS)], axis=0)
        t = jnp.concatenate([at.reshape(3 * MAX2D, CS),
                             rt.reshape(10 * MAX2D, CS)], axis=0)
        w = _pack_words(jnp.concatenate([v, t], axis=1))
        return jnp.pad(w, ((0, 0), (0, TW - w.shape[1])))

    tx = table(abs_v_x, rel_v_x, abs_t_x, rel_t_x)
    ty = table(abs_v_y, rel_v_y, abs_t_y, rel_t_y)
    xy = jnp.stack([x_feature.reshape(B, NSB, NIDX),
                    y_feature.reshape(B, NSB, NIDX)], axis=2)
    xy = xy.reshape(B * NSB * 2 * NIDX)
    pe = _pack_words(jnp.asarray(_pe_table(S, D)).reshape(S * NCHUNK, CS))

    mesh = plsc.VectorSubcoreMesh(core_axis_name="c", subcore_axis_name="s")
    f = functools.partial(
        pl.kernel,
        out_type=(jax.ShapeDtypeStruct((B, S, D), jnp.float32),
                  jax.ShapeDtypeStruct((B, S, D), jnp.float32)),
        mesh=mesh,
        compiler_params=pltpu.CompilerParams(use_tc_tiling_on_sc=True,
                                             needs_layout_passes=False),
        scratch_types=[
            pltpu.VMEM((2, 2 * NIDX), jnp.int32),
            pltpu.VMEM((2, NIDX), jnp.int32),
            pltpu.VMEM((2, NIDX), jnp.int32),
            pltpu.VMEM((2, NIDX, TW), jnp.int32),
            pltpu.VMEM((2, NIDX, TW), jnp.int32),
            pltpu.VMEM((2, W, D), jnp.float32),
            pltpu.VMEM((2, W, D), jnp.float32),
            pltpu.VMEM((NIDX, PW), jnp.int32),
            [pltpu.SemaphoreType.DMA] * 2,
            [pltpu.SemaphoreType.DMA] * 2,
            [pltpu.SemaphoreType.DMA] * 2,
            [pltpu.SemaphoreType.DMA] * 2,
            [pltpu.SemaphoreType.DMA] * 2,
        ],
    )(_sc_body)
    v_bar, t_bar = f(tx, ty, xy, pe)
    return (v_bar, t_bar)
